# baseline (speedup 1.0000x reference)
import dataclasses
import functools

import jax
import jax.numpy as jnp
from jax.experimental import pallas as pl
from jax.experimental.pallas import tpu as pltpu
from jax.experimental.pallas import tpu_sc as plsc

P_TGT = 64
K_SEL = 8
N_SEL = P_TGT * K_SEL
LANES = 16
ROWS_PER_BLOCK = 16


def _sc_gather(y2, idx_flat):
    n_rows, d = y2.shape
    mesh = plsc.VectorSubcoreMesh(
        core_axis_name="c", subcore_axis_name="s", num_cores=2, num_subcores=16
    )

    cp = pltpu.CompilerParams()
    if "needs_layout_passes" in pltpu.CompilerParams.__dataclass_fields__:
        cp = dataclasses.replace(cp, needs_layout_passes=False)

    @functools.partial(
        pl.kernel,
        out_type=jax.ShapeDtypeStruct((n_rows, N_SEL), jnp.float32),
        mesh=mesh,
        scratch_types=[pltpu.VMEM((N_SEL,), jnp.int32)],
        compiler_params=cp,
    )
    def k(y_hbm, idx_hbm, out_hbm, idx_v):
        pltpu.sync_copy(idx_hbm, idx_v)

        def body(in_v, out_v):
            for j in range(N_SEL // LANES):
                idx_j = idx_v[pl.ds(j * LANES, LANES)]
                for r in range(ROWS_PER_BLOCK):
                    row_sel = plsc.load_gather(
                        in_v, [jnp.full((LANES,), r, jnp.int32), idx_j]
                    )
                    out_v[r, pl.ds(j * LANES, LANES)] = row_sel

        pltpu.emit_pipeline(
            body,
            grid=(n_rows // ROWS_PER_BLOCK,),
            in_specs=[pl.BlockSpec((ROWS_PER_BLOCK, d), lambda i: (i, 0))],
            out_specs=[pl.BlockSpec((ROWS_PER_BLOCK, N_SEL), lambda i: (i, 0))],
            core_axis_name=("c", "s"),
            dimension_semantics=(pltpu.PARALLEL,),
        )(y_hbm, out_hbm)

    return k(y2, idx_flat)


def kernel(y, src_idx):
    B, T, D = y.shape
    y2 = y.reshape(B * T, D)
    idx_flat = src_idx.reshape(-1).astype(jnp.int32)
    out = _sc_gather(y2, idx_flat)
    return out.reshape(B, T, P_TGT, K_SEL)


# speedup vs baseline: 1.0641x; 1.0641x over previous
import dataclasses
import functools

import jax
import jax.numpy as jnp
from jax.experimental import pallas as pl
from jax.experimental.pallas import tpu as pltpu
from jax.experimental.pallas import tpu_sc as plsc

P_TGT = 64
K_SEL = 8
N_SEL = P_TGT * K_SEL
LANES = 16
ROWS_PER_BLOCK = 8


def _sc_gather(y2, idx_flat):
    n_rows, d = y2.shape
    mesh = plsc.VectorSubcoreMesh(
        core_axis_name="c", subcore_axis_name="s", num_cores=2, num_subcores=16
    )

    cp = pltpu.CompilerParams(use_tc_tiling_on_sc=True)
    if "needs_layout_passes" in pltpu.CompilerParams.__dataclass_fields__:
        cp = dataclasses.replace(cp, needs_layout_passes=False)

    @functools.partial(
        pl.kernel,
        out_type=jax.ShapeDtypeStruct((n_rows, N_SEL), jnp.float32),
        mesh=mesh,
        scratch_types=[pltpu.VMEM((N_SEL,), jnp.int32)],
        compiler_params=cp,
    )
    def k(y_hbm, idx_hbm, out_hbm, idx_v):
        pltpu.sync_copy(idx_hbm, idx_v)

        def body(in_v, out_v):
            for j in range(N_SEL // LANES):
                idx_j = idx_v[pl.ds(j * LANES, LANES)]
                for r in range(ROWS_PER_BLOCK):
                    row_sel = plsc.load_gather(
                        in_v, [jnp.full((LANES,), r, jnp.int32), idx_j]
                    )
                    out_v[r, pl.ds(j * LANES, LANES)] = row_sel

        pltpu.emit_pipeline(
            body,
            grid=(n_rows // ROWS_PER_BLOCK,),
            in_specs=[pl.BlockSpec((ROWS_PER_BLOCK, d), lambda i: (i, 0))],
            out_specs=[pl.BlockSpec((ROWS_PER_BLOCK, N_SEL), lambda i: (i, 0))],
            core_axis_name=("c", "s"),
            dimension_semantics=(pltpu.PARALLEL,),
        )(y_hbm, out_hbm)

    return k(y2, idx_flat)


def kernel(y, src_idx):
    B, T, D = y.shape
    y2 = y.reshape(B * T, D)
    idx_flat = src_idx.reshape(-1).astype(jnp.int32)
    out = _sc_gather(y2, idx_flat)
    return out.reshape(B, T, P_TGT, K_SEL)
